# Initial kernel scaffold; baseline (speedup 1.0000x reference)
import jax
import jax.numpy as jnp
from jax.experimental import pallas as pl


def kernel(z, W):
    raise NotImplementedError("write your pallas kernel here")


# speedup vs baseline: 1.3498x; 1.3498x over previous
import functools

import jax
import jax.numpy as jnp
from jax import lax
from jax.experimental import pallas as pl
from jax.experimental.pallas import tpu as pltpu
from jax.experimental.pallas import tpu_sc as plsc

BLK = 1024
CHUNK = 128


def _tc_body(z_ref, w_ref, idx_ref, loss_ref):
    i = pl.program_id(0)
    z = z_ref[...]
    w = w_ref[...]
    e = w.shape[0]
    s = lax.dot_general(z, w, (((1,), (1,)), ((), ())),
                        preferred_element_type=jnp.float32)
    zsq = jnp.sum(z ** 2, axis=1, keepdims=True)
    wsq = jnp.sum(w ** 2, axis=1)
    d = (zsq - 2.0 * s) + wsq[None, :]
    dmin = jnp.min(d, axis=1, keepdims=True)
    iota = lax.broadcasted_iota(jnp.int32, d.shape, 1)
    idx = jnp.min(jnp.where(d == dmin, iota, e), axis=1)
    idx_ref[...] = idx.reshape(1, 1, idx.shape[0])

    @pl.when(i == 0)
    def _():
        loss_ref[...] = jnp.zeros_like(loss_ref)

    loss_ref[...] += jnp.sum(dmin)


def _distance_argmin(z, W):
    n, d = z.shape
    e = W.shape[0]
    nblk = n // BLK
    return pl.pallas_call(
        _tc_body,
        grid=(nblk,),
        in_specs=[
            pl.BlockSpec((BLK, d), lambda i: (i, 0)),
            pl.BlockSpec((e, d), lambda i: (0, 0)),
        ],
        out_specs=[
            pl.BlockSpec((1, 1, BLK), lambda i: (i, 0, 0)),
            pl.BlockSpec((1, 1), lambda i: (0, 0)),
        ],
        out_shape=[
            jax.ShapeDtypeStruct((nblk, 1, BLK), jnp.int32),
            jax.ShapeDtypeStruct((1, 1), jnp.float32),
        ],
    )(z, W)


T = 512


def _sc_gather(w_flat, idx_flat, d):
    n = idx_flat.shape[0]
    ed = w_flat.shape[0]
    info = plsc.get_sparse_core_info()
    nw = info.num_cores * info.num_subcores
    tok_per_w = n // nw
    nchunks = tok_per_w // T
    mesh = plsc.VectorSubcoreMesh(core_axis_name="c", subcore_axis_name="s")

    @functools.partial(
        pl.kernel,
        out_type=jax.ShapeDtypeStruct((n * d,), jnp.float32),
        mesh=mesh,
        compiler_params=pltpu.CompilerParams(needs_layout_passes=False),
        scratch_types=[
            pltpu.VMEM((ed,), jnp.float32),
            pltpu.VMEM((T,), jnp.int32),
            pltpu.VMEM((T * d,), jnp.float32),
        ],
    )
    def k(w_hbm, idx_hbm, out_hbm, w_v, idx_v, out_v):
        wid = lax.axis_index("s") * info.num_cores + lax.axis_index("c")
        tok0 = wid * tok_per_w
        pltpu.sync_copy(w_hbm, w_v)
        iota16 = lax.iota(jnp.int32, 16)

        def chunk_body(kk, carry):
            pltpu.sync_copy(idx_hbm.at[pl.ds(tok0 + kk * T, T)], idx_v)

            def group_body(g, carry2):
                iv = idx_v[pl.ds(g * 16, 16)]
                addr = iv * d
                pos = iota16 * d + g * (16 * d)
                for _ in range(d):
                    v = plsc.load_gather(w_v, [addr])
                    plsc.store_scatter(out_v, [pos], v)
                    addr = addr + 1
                    pos = pos + 1
                return carry2

            lax.fori_loop(0, T // 16, group_body, 0, unroll=False)
            pltpu.sync_copy(out_v,
                            out_hbm.at[pl.ds((tok0 + kk * T) * d, T * d)])
            return carry

        lax.fori_loop(0, nchunks, chunk_body, 0, unroll=False)

    return k(w_flat, idx_flat)


def kernel(z, W):
    n, d = z.shape
    idx3, loss_sum = _distance_argmin(z, W)
    idx = idx3.reshape(n)
    zq_flat = _sc_gather(W.reshape(-1), idx, d)
    zq = zq_flat.reshape(n, d)
    loss = 1.25 * loss_sum[0, 0] / (n * d)
    return (zq, idx, loss)


# speedup vs baseline: 1.5009x; 1.1119x over previous
import functools

import jax
import jax.numpy as jnp
from jax import lax
from jax.experimental import pallas as pl
from jax.experimental.pallas import tpu as pltpu
from jax.experimental.pallas import tpu_sc as plsc

BLK = 1024
CHUNK = 128


def _tc_body(z_ref, w_ref, idx_ref, loss_ref):
    i = pl.program_id(0)
    z = z_ref[...]
    w = w_ref[...]
    e = w.shape[0]
    s = lax.dot_general(z, w, (((1,), (1,)), ((), ())),
                        preferred_element_type=jnp.float32)
    zsq = jnp.sum(z ** 2, axis=1, keepdims=True)
    wsq = jnp.sum(w ** 2, axis=1)
    d = (zsq - 2.0 * s) + wsq[None, :]
    dmin = jnp.min(d, axis=1, keepdims=True)
    iota = lax.broadcasted_iota(jnp.int32, d.shape, 1)
    idx = jnp.min(jnp.where(d == dmin, iota, e), axis=1)
    idx_ref[...] = idx.reshape(1, 1, idx.shape[0])

    @pl.when(i == 0)
    def _():
        loss_ref[...] = jnp.zeros_like(loss_ref)

    loss_ref[...] += jnp.sum(dmin)


def _distance_argmin(z, W):
    n, d = z.shape
    e = W.shape[0]
    nblk = n // BLK
    return pl.pallas_call(
        _tc_body,
        grid=(nblk,),
        in_specs=[
            pl.BlockSpec((BLK, d), lambda i: (i, 0)),
            pl.BlockSpec((e, d), lambda i: (0, 0)),
        ],
        out_specs=[
            pl.BlockSpec((1, 1, BLK), lambda i: (i, 0, 0)),
            pl.BlockSpec((1, 1), lambda i: (0, 0)),
        ],
        out_shape=[
            jax.ShapeDtypeStruct((nblk, 1, BLK), jnp.int32),
            jax.ShapeDtypeStruct((1, 1), jnp.float32),
        ],
    )(z, W)


T = 512


def _sc_gather(w_flat, idx_flat, d):
    n = idx_flat.shape[0]
    ed = w_flat.shape[0]
    info = plsc.get_sparse_core_info()
    nw = info.num_cores * info.num_subcores
    tok_per_w = n // nw
    nchunks = tok_per_w // T
    mesh = plsc.VectorSubcoreMesh(core_axis_name="c", subcore_axis_name="s")

    @functools.partial(
        pl.kernel,
        out_type=jax.ShapeDtypeStruct((n * d,), jnp.float32),
        mesh=mesh,
        compiler_params=pltpu.CompilerParams(needs_layout_passes=False),
        scratch_types=[
            pltpu.VMEM((ed,), jnp.float32),
            pltpu.VMEM((T,), jnp.int32),
            pltpu.VMEM((T * d,), jnp.float32),
        ],
    )
    def k(w_hbm, idx_hbm, out_hbm, w_v, idx_v, out_v):
        wid = lax.axis_index("s") * info.num_cores + lax.axis_index("c")
        tok0 = wid * tok_per_w
        pltpu.sync_copy(w_hbm, w_v)
        iota16 = lax.iota(jnp.int32, 16)

        def chunk_body(kk, carry):
            pltpu.sync_copy(idx_hbm.at[pl.ds(tok0 + kk * T, T)], idx_v)

            @plsc.parallel_loop(0, T // 16, 1, unroll=4)
            def group_body(g):
                iv = idx_v[pl.ds(g * 16, 16)]
                addr = iv * d
                pos = iota16 * d + g * (16 * d)
                for c in range(d):
                    v = plsc.load_gather(w_v, [addr + c])
                    plsc.store_scatter(out_v, [pos + c], v)

            pltpu.sync_copy(out_v,
                            out_hbm.at[pl.ds((tok0 + kk * T) * d, T * d)])
            return carry

        lax.fori_loop(0, nchunks, chunk_body, 0, unroll=False)

    return k(w_flat, idx_flat)


def kernel(z, W):
    n, d = z.shape
    idx3, loss_sum = _distance_argmin(z, W)
    idx = idx3.reshape(n)
    zq_flat = _sc_gather(W.reshape(-1), idx, d)
    zq = zq_flat.reshape(n, d)
    loss = 1.25 * loss_sum[0, 0] / (n * d)
    return (zq, idx, loss)


# speedup vs baseline: 1.7838x; 1.1885x over previous
import functools

import jax
import jax.numpy as jnp
from jax import lax
from jax.experimental import pallas as pl
from jax.experimental.pallas import tpu as pltpu
from jax.experimental.pallas import tpu_sc as plsc

BLK = 1024
CHUNK = 128


def _tc_body(z_ref, w_ref, idx_ref, loss_ref):
    i = pl.program_id(0)
    z = z_ref[...]
    w = w_ref[...]
    e = w.shape[0]
    blk = z.shape[0]
    s = lax.dot_general(w, z, (((1,), (1,)), ((), ())),
                        preferred_element_type=jnp.float32)
    zsq = jnp.sum(z * z, axis=1, keepdims=True).T
    wsq = jnp.sum(w * w, axis=1, keepdims=True)
    d = (zsq - 2.0 * s) + wsq
    dmin = jnp.min(d, axis=0, keepdims=True)
    iota = lax.broadcasted_iota(jnp.int32, d.shape, 0)
    idx = jnp.min(jnp.where(d == dmin, iota, e), axis=0)
    idx_ref[...] = idx.reshape(1, 1, blk)

    @pl.when(i == 0)
    def _():
        loss_ref[...] = jnp.zeros_like(loss_ref)

    loss_ref[...] += jnp.sum(dmin)


def _distance_argmin(z, W):
    n, d = z.shape
    e = W.shape[0]
    nblk = n // BLK
    return pl.pallas_call(
        _tc_body,
        grid=(nblk,),
        in_specs=[
            pl.BlockSpec((BLK, d), lambda i: (i, 0)),
            pl.BlockSpec((e, d), lambda i: (0, 0)),
        ],
        out_specs=[
            pl.BlockSpec((1, 1, BLK), lambda i: (i, 0, 0)),
            pl.BlockSpec((1, 1), lambda i: (0, 0)),
        ],
        out_shape=[
            jax.ShapeDtypeStruct((nblk, 1, BLK), jnp.int32),
            jax.ShapeDtypeStruct((1, 1), jnp.float32),
        ],
    )(z, W)


T = 512


def _sc_gather(w_flat, idx_flat, d):
    n = idx_flat.shape[0]
    ed = w_flat.shape[0]
    info = plsc.get_sparse_core_info()
    nw = info.num_cores * info.num_subcores
    tok_per_w = n // nw
    nchunks = tok_per_w // T
    mesh = plsc.VectorSubcoreMesh(core_axis_name="c", subcore_axis_name="s")

    @functools.partial(
        pl.kernel,
        out_type=jax.ShapeDtypeStruct((n * d,), jnp.float32),
        mesh=mesh,
        compiler_params=pltpu.CompilerParams(needs_layout_passes=False),
        scratch_types=[
            pltpu.VMEM((ed,), jnp.float32),
            pltpu.VMEM((T,), jnp.int32),
            pltpu.VMEM((T * d,), jnp.float32),
        ],
    )
    def k(w_hbm, idx_hbm, out_hbm, w_v, idx_v, out_v):
        wid = lax.axis_index("s") * info.num_cores + lax.axis_index("c")
        tok0 = wid * tok_per_w
        pltpu.sync_copy(w_hbm, w_v)
        iota16 = lax.iota(jnp.int32, 16)

        def chunk_body(kk, carry):
            pltpu.sync_copy(idx_hbm.at[pl.ds(tok0 + kk * T, T)], idx_v)

            @plsc.parallel_loop(0, T // 16, 1, unroll=4)
            def group_body(g):
                iv = idx_v[pl.ds(g * 16, 16)]
                addr = iv * d
                pos = iota16 * d + g * (16 * d)
                for c in range(d):
                    v = plsc.load_gather(w_v, [addr + c])
                    plsc.store_scatter(out_v, [pos + c], v)

            pltpu.sync_copy(out_v,
                            out_hbm.at[pl.ds((tok0 + kk * T) * d, T * d)])
            return carry

        lax.fori_loop(0, nchunks, chunk_body, 0, unroll=False)

    return k(w_flat, idx_flat)


def kernel(z, W):
    n, d = z.shape
    idx3, loss_sum = _distance_argmin(z, W)
    idx = idx3.reshape(n)
    zq_flat = _sc_gather(W.reshape(-1), idx, d)
    zq = zq_flat.reshape(n, d)
    loss = 1.25 * loss_sum[0, 0] / (n * d)
    return (zq, idx, loss)


# speedup vs baseline: 1.8250x; 1.0230x over previous
import functools

import jax
import jax.numpy as jnp
from jax import lax
from jax.experimental import pallas as pl
from jax.experimental.pallas import tpu as pltpu
from jax.experimental.pallas import tpu_sc as plsc

BLK = 1024
CHUNK = 128


def _tc_body(z_ref, w_ref, idx_ref, loss_ref):
    i = pl.program_id(0)
    z = z_ref[...]
    w = w_ref[...]
    e = w.shape[0]
    blk = z.shape[0]
    s = lax.dot_general(w, z, (((1,), (1,)), ((), ())),
                        preferred_element_type=jnp.float32)
    zsq = jnp.sum(z * z, axis=1, keepdims=True).T
    wsq = jnp.sum(w * w, axis=1, keepdims=True)
    d = (zsq - 2.0 * s) + wsq
    dmin = jnp.min(d, axis=0, keepdims=True)
    iota = lax.broadcasted_iota(jnp.int32, d.shape, 0)
    idx = jnp.min(jnp.where(d == dmin, iota, e), axis=0)
    idx_ref[...] = idx.reshape(1, 1, blk)

    @pl.when(i == 0)
    def _():
        loss_ref[...] = jnp.zeros_like(loss_ref)

    loss_ref[...] += jnp.sum(dmin)


def _distance_argmin(z, W):
    n, d = z.shape
    e = W.shape[0]
    nblk = n // BLK
    return pl.pallas_call(
        _tc_body,
        grid=(nblk,),
        in_specs=[
            pl.BlockSpec((BLK, d), lambda i: (i, 0)),
            pl.BlockSpec((e, d), lambda i: (0, 0)),
        ],
        out_specs=[
            pl.BlockSpec((1, 1, BLK), lambda i: (i, 0, 0)),
            pl.BlockSpec((1, 1), lambda i: (0, 0)),
        ],
        out_shape=[
            jax.ShapeDtypeStruct((nblk, 1, BLK), jnp.int32),
            jax.ShapeDtypeStruct((1, 1), jnp.float32),
        ],
    )(z, W)


T = 512


def _sc_gather(w_flat, idx_flat, d):
    n = idx_flat.shape[0]
    ed = w_flat.shape[0]
    info = plsc.get_sparse_core_info()
    nw = info.num_cores * info.num_subcores
    tok_per_w = n // nw
    nchunks = tok_per_w // T
    mesh = plsc.VectorSubcoreMesh(core_axis_name="c", subcore_axis_name="s")

    @functools.partial(
        pl.kernel,
        out_type=jax.ShapeDtypeStruct((n, d), jnp.float32),
        mesh=mesh,
        compiler_params=pltpu.CompilerParams(needs_layout_passes=False),
        scratch_types=[
            pltpu.VMEM((ed,), jnp.float32),
            pltpu.VMEM((T,), jnp.int32),
            pltpu.VMEM((T, d), jnp.float32),
        ],
    )
    def k(w_hbm, idx_hbm, out_hbm, w_v, idx_v, out_v):
        wid = lax.axis_index("s") * info.num_cores + lax.axis_index("c")
        tok0 = wid * tok_per_w
        pltpu.sync_copy(w_hbm, w_v)
        iota16 = lax.iota(jnp.int32, 16)

        def chunk_body(kk, carry):
            pltpu.sync_copy(idx_hbm.at[pl.ds(tok0 + kk * T, T)], idx_v)

            @plsc.parallel_loop(0, T // 16, 1, unroll=2)
            def group_body(g):
                iv = idx_v[pl.ds(g * 16, 16)]
                addr = iv * d
                rows = iota16 + g * 16
                vals = [plsc.load_gather(w_v, [addr + c]) for c in range(d)]
                for c in range(d):
                    plsc.store_scatter(out_v, [rows, iota16 * 0 + c], vals[c])

            pltpu.sync_copy(out_v, out_hbm.at[pl.ds(tok0 + kk * T, T)])
            return carry

        lax.fori_loop(0, nchunks, chunk_body, 0, unroll=False)

    return k(w_flat, idx_flat)


def kernel(z, W):
    n, d = z.shape
    idx3, loss_sum = _distance_argmin(z, W)
    idx = idx3.reshape(n)
    zq = _sc_gather(W.reshape(-1), idx, d)
    loss = 1.25 * loss_sum[0, 0] / (n * d)
    return (zq, idx, loss)


# speedup vs baseline: 1.9860x; 1.0883x over previous
import functools

import jax
import jax.numpy as jnp
from jax import lax
from jax.experimental import pallas as pl
from jax.experimental.pallas import tpu as pltpu
from jax.experimental.pallas import tpu_sc as plsc

BLK = 1024
CHUNK = 128


def _tc_body(z_ref, w_ref, idx_ref, loss_ref):
    i = pl.program_id(0)
    z = z_ref[...]
    w = w_ref[...]
    e = w.shape[0]
    blk = z.shape[0]
    s = lax.dot_general(w, z, (((1,), (1,)), ((), ())),
                        preferred_element_type=jnp.float32)
    zsq = jnp.sum(z * z, axis=1, keepdims=True).T
    wsq = jnp.sum(w * w, axis=1, keepdims=True)
    d = (zsq - 2.0 * s) + wsq
    dmin = jnp.min(d, axis=0, keepdims=True)
    iota = lax.broadcasted_iota(jnp.int32, d.shape, 0)
    idx = jnp.min(jnp.where(d == dmin, iota, e), axis=0)
    idx_ref[...] = idx.reshape(1, 1, blk)

    @pl.when(i == 0)
    def _():
        loss_ref[...] = jnp.zeros_like(loss_ref)

    loss_ref[...] += jnp.sum(dmin)


def _distance_argmin(z, W):
    n, d = z.shape
    e = W.shape[0]
    nblk = n // BLK
    return pl.pallas_call(
        _tc_body,
        grid=(nblk,),
        in_specs=[
            pl.BlockSpec((BLK, d), lambda i: (i, 0)),
            pl.BlockSpec((e, d), lambda i: (0, 0)),
        ],
        out_specs=[
            pl.BlockSpec((1, 1, BLK), lambda i: (i, 0, 0)),
            pl.BlockSpec((1, 1), lambda i: (0, 0)),
        ],
        out_shape=[
            jax.ShapeDtypeStruct((nblk, 1, BLK), jnp.int32),
            jax.ShapeDtypeStruct((1, 1), jnp.float32),
        ],
    )(z, W)


T = 256


def _sc_gather(w_flat, idx_flat, d):
    n = idx_flat.shape[0]
    ed = w_flat.shape[0]
    info = plsc.get_sparse_core_info()
    nw = info.num_cores * info.num_subcores
    tok_per_w = n // nw
    nchunks = tok_per_w // T
    mesh = plsc.VectorSubcoreMesh(core_axis_name="c", subcore_axis_name="s")

    @functools.partial(
        pl.kernel,
        out_type=jax.ShapeDtypeStruct((n, d), jnp.float32),
        mesh=mesh,
        compiler_params=pltpu.CompilerParams(needs_layout_passes=False),
        scratch_types=[
            pltpu.VMEM((ed,), jnp.float32),
            pltpu.VMEM((T,), jnp.int32),
            pltpu.VMEM((T, d), jnp.float32),
            pltpu.VMEM((T, d), jnp.float32),
            pltpu.SemaphoreType.DMA,
            pltpu.SemaphoreType.DMA,
        ],
    )
    def k(w_hbm, idx_hbm, out_hbm, w_v, idx_v, ob0, ob1, sem0, sem1):
        wid = lax.axis_index("s") * info.num_cores + lax.axis_index("c")
        tok0 = wid * tok_per_w
        pltpu.sync_copy(w_hbm, w_v)
        iota16 = lax.iota(jnp.int32, 16)
        obs = (ob0, ob1)
        sems = (sem0, sem1)

        def pair_body(p, carry):
            for b in range(2):
                kk = 2 * p + b
                out_v = obs[b]

                pltpu.sync_copy(idx_hbm.at[pl.ds(tok0 + kk * T, T)], idx_v)

                @pl.when(kk >= 2)
                def _():
                    pltpu.make_async_copy(
                        out_v, out_hbm.at[pl.ds(tok0, T)], sems[b]).wait()

                @plsc.parallel_loop(0, T // 16, 1, unroll=2)
                def group_body(g):
                    iv = idx_v[pl.ds(g * 16, 16)]
                    addr = iv * d
                    rows = iota16 + g * 16
                    for c0 in range(0, d, 8):
                        vals = [plsc.load_gather(w_v, [addr + (c0 + j)])
                                for j in range(8)]
                        for j in range(8):
                            plsc.store_scatter(
                                out_v, [rows, iota16 * 0 + (c0 + j)], vals[j])

                pltpu.async_copy(
                    out_v, out_hbm.at[pl.ds(tok0 + kk * T, T)], sems[b])
            return carry

        lax.fori_loop(0, nchunks // 2, pair_body, 0, unroll=False)
        for b in range(2):
            pltpu.make_async_copy(
                obs[b], out_hbm.at[pl.ds(tok0, T)], sems[b]).wait()

    return k(w_flat, idx_flat)


def kernel(z, W):
    n, d = z.shape
    idx3, loss_sum = _distance_argmin(z, W)
    idx = idx3.reshape(n)
    zq = _sc_gather(W.reshape(-1), idx, d)
    loss = 1.25 * loss_sum[0, 0] / (n * d)
    return (zq, idx, loss)


# speedup vs baseline: 2.0522x; 1.0333x over previous
import functools

import jax
import jax.numpy as jnp
from jax import lax
from jax.experimental import pallas as pl
from jax.experimental.pallas import tpu as pltpu
from jax.experimental.pallas import tpu_sc as plsc

BLK = 1024
CHUNK = 128


def _tc_body(z_ref, w_ref, idx_ref, loss_ref):
    i = pl.program_id(0)
    z = z_ref[...]
    w = w_ref[...]
    e = w.shape[0]
    blk = z.shape[0]
    s = lax.dot_general(w, z, (((1,), (1,)), ((), ())),
                        preferred_element_type=jnp.float32)
    zsq = jnp.sum(z * z, axis=1, keepdims=True).T
    wsq = jnp.sum(w * w, axis=1, keepdims=True)
    d = (zsq - 2.0 * s) + wsq
    dmin = jnp.min(d, axis=0, keepdims=True)
    iota = lax.broadcasted_iota(jnp.int32, d.shape, 0)
    idx = jnp.min(jnp.where(d == dmin, iota, e), axis=0)
    idx_ref[...] = idx.reshape(1, 1, blk)

    @pl.when(i == 0)
    def _():
        loss_ref[...] = jnp.zeros_like(loss_ref)

    loss_ref[...] += jnp.sum(dmin)


def _distance_argmin(z, W):
    n, d = z.shape
    e = W.shape[0]
    nblk = n // BLK
    return pl.pallas_call(
        _tc_body,
        grid=(nblk,),
        in_specs=[
            pl.BlockSpec((BLK, d), lambda i: (i, 0)),
            pl.BlockSpec((e, d), lambda i: (0, 0)),
        ],
        out_specs=[
            pl.BlockSpec((1, 1, BLK), lambda i: (i, 0, 0)),
            pl.BlockSpec((1, 1), lambda i: (0, 0)),
        ],
        out_shape=[
            jax.ShapeDtypeStruct((nblk, 1, BLK), jnp.int32),
            jax.ShapeDtypeStruct((1, 1), jnp.float32),
        ],
    )(z, W)


T = 256


def _sc_gather(w_flat, idx_flat, d):
    n = idx_flat.shape[0]
    ed = w_flat.shape[0]
    info = plsc.get_sparse_core_info()
    nw = info.num_cores * info.num_subcores
    tok_per_w = n // nw
    nchunks = tok_per_w // T
    mesh = plsc.VectorSubcoreMesh(core_axis_name="c", subcore_axis_name="s")

    @functools.partial(
        pl.kernel,
        out_type=jax.ShapeDtypeStruct((n, d), jnp.float32),
        mesh=mesh,
        compiler_params=pltpu.CompilerParams(needs_layout_passes=False),
        scratch_types=[
            pltpu.VMEM((ed,), jnp.float32),
            pltpu.VMEM((T,), jnp.int32),
            pltpu.VMEM((T, d), jnp.float32),
            pltpu.VMEM((T, d), jnp.float32),
            pltpu.SemaphoreType.DMA,
            pltpu.SemaphoreType.DMA,
        ],
    )
    def k(w_hbm, idx_hbm, out_hbm, w_v, idx_v, ob0, ob1, sem0, sem1):
        wid = lax.axis_index("s") * info.num_cores + lax.axis_index("c")
        tok0 = wid * tok_per_w
        pltpu.sync_copy(w_hbm, w_v)
        iota16 = lax.iota(jnp.int32, 16)
        obs = (ob0, ob1)
        sems = (sem0, sem1)

        def pair_body(p, carry):
            for b in range(2):
                kk = 2 * p + b
                out_v = obs[b]

                pltpu.sync_copy(idx_hbm.at[pl.ds(tok0 + kk * T, T)], idx_v)

                @pl.when(kk >= 2)
                def _():
                    pltpu.make_async_copy(
                        out_v, out_hbm.at[pl.ds(tok0, T)], sems[b]).wait()

                @plsc.parallel_loop(0, T // 16, 1, unroll=4)
                def group_body(g):
                    iv = idx_v[pl.ds(g * 16, 16)]
                    addr = iv * d
                    rows = iota16 + g * 16
                    for c0 in range(0, d, 8):
                        vals = [plsc.load_gather(w_v, [addr + (c0 + j)])
                                for j in range(8)]
                        for j in range(8):
                            plsc.store_scatter(
                                out_v, [rows, iota16 * 0 + (c0 + j)], vals[j])

                pltpu.async_copy(
                    out_v, out_hbm.at[pl.ds(tok0 + kk * T, T)], sems[b])
            return carry

        lax.fori_loop(0, nchunks // 2, pair_body, 0, unroll=False)
        for b in range(2):
            pltpu.make_async_copy(
                obs[b], out_hbm.at[pl.ds(tok0, T)], sems[b]).wait()

    return k(w_flat, idx_flat)


def kernel(z, W):
    n, d = z.shape
    idx3, loss_sum = _distance_argmin(z, W)
    idx = idx3.reshape(n)
    zq = _sc_gather(W.reshape(-1), idx, d)
    loss = 1.25 * loss_sum[0, 0] / (n * d)
    return (zq, idx, loss)


# speedup vs baseline: 2.1596x; 1.0523x over previous
import functools

import jax
import jax.numpy as jnp
from jax import lax
from jax.experimental import pallas as pl
from jax.experimental.pallas import tpu as pltpu
from jax.experimental.pallas import tpu_sc as plsc

BLK = 2048
CHUNK = 128


def _tc_body(z_ref, w_ref, idx_ref, loss_ref):
    i = pl.program_id(0)
    z = z_ref[...]
    w = w_ref[...]
    e = w.shape[0]
    blk = z.shape[0]
    s = lax.dot_general(w, z, (((1,), (1,)), ((), ())),
                        preferred_element_type=jnp.float32)
    zsq = jnp.sum(z * z, axis=1, keepdims=True).T
    wsq = jnp.sum(w * w, axis=1, keepdims=True)
    d = (zsq - 2.0 * s) + wsq
    dmin = jnp.min(d, axis=0, keepdims=True)
    iota = lax.broadcasted_iota(jnp.int32, d.shape, 0)
    idx = jnp.min(jnp.where(d == dmin, iota, e), axis=0)
    idx_ref[...] = idx.reshape(1, 1, blk)

    @pl.when(i == 0)
    def _():
        loss_ref[...] = jnp.zeros_like(loss_ref)

    loss_ref[...] += jnp.sum(dmin)


def _distance_argmin(z, W):
    n, d = z.shape
    e = W.shape[0]
    nblk = n // BLK
    return pl.pallas_call(
        _tc_body,
        grid=(nblk,),
        in_specs=[
            pl.BlockSpec((BLK, d), lambda i: (i, 0)),
            pl.BlockSpec((e, d), lambda i: (0, 0)),
        ],
        out_specs=[
            pl.BlockSpec((1, 1, BLK), lambda i: (i, 0, 0)),
            pl.BlockSpec((1, 1), lambda i: (0, 0)),
        ],
        out_shape=[
            jax.ShapeDtypeStruct((nblk, 1, BLK), jnp.int32),
            jax.ShapeDtypeStruct((1, 1), jnp.float32),
        ],
    )(z, W)


T = 256


def _sc_gather(w_flat, idx_flat, d):
    n = idx_flat.shape[0]
    ed = w_flat.shape[0]
    info = plsc.get_sparse_core_info()
    nw = info.num_cores * info.num_subcores
    tok_per_w = n // nw
    nchunks = tok_per_w // T
    mesh = plsc.VectorSubcoreMesh(core_axis_name="c", subcore_axis_name="s")

    @functools.partial(
        pl.kernel,
        out_type=jax.ShapeDtypeStruct((n, d), jnp.float32),
        mesh=mesh,
        compiler_params=pltpu.CompilerParams(needs_layout_passes=False),
        scratch_types=[
            pltpu.VMEM((ed,), jnp.float32),
            pltpu.VMEM((T,), jnp.int32),
            pltpu.VMEM((T, d), jnp.float32),
            pltpu.VMEM((T, d), jnp.float32),
            pltpu.SemaphoreType.DMA,
            pltpu.SemaphoreType.DMA,
        ],
    )
    def k(w_hbm, idx_hbm, out_hbm, w_v, idx_v, ob0, ob1, sem0, sem1):
        wid = lax.axis_index("s") * info.num_cores + lax.axis_index("c")
        tok0 = wid * tok_per_w
        pltpu.sync_copy(w_hbm, w_v)
        iota16 = lax.iota(jnp.int32, 16)
        obs = (ob0, ob1)
        sems = (sem0, sem1)

        def pair_body(p, carry):
            for b in range(2):
                kk = 2 * p + b
                out_v = obs[b]

                pltpu.sync_copy(idx_hbm.at[pl.ds(tok0 + kk * T, T)], idx_v)

                @pl.when(kk >= 2)
                def _():
                    pltpu.make_async_copy(
                        out_v, out_hbm.at[pl.ds(tok0, T)], sems[b]).wait()

                @plsc.parallel_loop(0, T // 16, 1, unroll=4)
                def group_body(g):
                    iv = idx_v[pl.ds(g * 16, 16)]
                    addr = iv * d
                    rows = iota16 + g * 16
                    for c0 in range(0, d, 8):
                        vals = [plsc.load_gather(w_v, [addr + (c0 + j)])
                                for j in range(8)]
                        for j in range(8):
                            plsc.store_scatter(
                                out_v, [rows, iota16 * 0 + (c0 + j)], vals[j])

                pltpu.async_copy(
                    out_v, out_hbm.at[pl.ds(tok0 + kk * T, T)], sems[b])
            return carry

        lax.fori_loop(0, nchunks // 2, pair_body, 0, unroll=False)
        for b in range(2):
            pltpu.make_async_copy(
                obs[b], out_hbm.at[pl.ds(tok0, T)], sems[b]).wait()

    return k(w_flat, idx_flat)


def kernel(z, W):
    n, d = z.shape
    idx3, loss_sum = _distance_argmin(z, W)
    idx = idx3.reshape(n)
    zq = _sc_gather(W.reshape(-1), idx, d)
    loss = 1.25 * loss_sum[0, 0] / (n * d)
    return (zq, idx, loss)
